# baseline (speedup 1.0000x reference)
import functools

import jax
import jax.numpy as jnp
from jax import lax
from jax.experimental import pallas as pl
from jax.experimental.pallas import tpu as pltpu
from jax.experimental.pallas import tpu_sc as plsc

_NUM_EXPERTS = 8
_TOP_K = 2
_LANES = 16


def _router_tc_body(x_ref, w_ref, logits_ref, scores_ref, scores_t_ref):
    l = lax.dot_general(
        x_ref[...], w_ref[...],
        dimension_numbers=(((1,), (1,)), ((), ())),
        preferred_element_type=jnp.float32,
    )
    logits_ref[...] = l
    m = jnp.max(l, axis=-1, keepdims=True)
    e = jnp.exp(l - m)
    s = e / jnp.sum(e, axis=-1, keepdims=True)
    scores_ref[...] = s
    scores_t_ref[...] = s.T


@functools.lru_cache(maxsize=None)
def _make_topk_sc(total_tokens):
    info = plsc.get_sparse_core_info()
    nc, ns = info.num_cores, info.num_subcores
    nw = nc * ns
    tpw = total_tokens // nw
    assert tpw * nw == total_tokens and tpw % _LANES == 0
    mesh = plsc.VectorSubcoreMesh(core_axis_name="c", subcore_axis_name="s")

    @functools.partial(
        pl.kernel,
        mesh=mesh,
        out_type=(
            jax.ShapeDtypeStruct((total_tokens,), jnp.float32),
            jax.ShapeDtypeStruct((total_tokens,), jnp.float32),
            jax.ShapeDtypeStruct((total_tokens,), jnp.int32),
            jax.ShapeDtypeStruct((total_tokens,), jnp.int32),
        ),
        scratch_types=[
            pltpu.VMEM((_NUM_EXPERTS * tpw,), jnp.float32),
            pltpu.VMEM((tpw,), jnp.float32),
            pltpu.VMEM((tpw,), jnp.float32),
            pltpu.VMEM((tpw,), jnp.int32),
            pltpu.VMEM((tpw,), jnp.int32),
        ],
    )
    def topk_sc(st_hbm, w1_hbm, w2_hbm, i1_hbm, i2_hbm, sv, wv1, wv2, iv1, iv2):
        wid = lax.axis_index("s") * nc + lax.axis_index("c")
        base = wid * tpw
        for e in range(_NUM_EXPERTS):
            pltpu.sync_copy(st_hbm.at[e, pl.ds(base, tpw)],
                            sv.at[pl.ds(e * tpw, tpw)])

        def body(g, carry):
            off = g * _LANES
            cols = [sv[pl.ds(e * tpw + off, _LANES)]
                    for e in range(_NUM_EXPERTS)]
            m1 = cols[0]
            i1 = jnp.zeros((_LANES,), jnp.int32)
            m2 = jnp.full((_LANES,), -jnp.inf, jnp.float32)
            i2 = jnp.zeros((_LANES,), jnp.int32)
            for e in range(1, _NUM_EXPERTS):
                c = cols[e]
                ev = jnp.full((_LANES,), e, jnp.int32)
                new_top = c > m1
                beats2 = c > m2
                m2 = jnp.where(new_top, m1, jnp.where(beats2, c, m2))
                i2 = jnp.where(new_top, i1, jnp.where(beats2, ev, i2))
                m1 = jnp.where(new_top, c, m1)
                i1 = jnp.where(new_top, ev, i1)
            wv1[pl.ds(off, _LANES)] = m1
            wv2[pl.ds(off, _LANES)] = m2
            iv1[pl.ds(off, _LANES)] = i1
            iv2[pl.ds(off, _LANES)] = i2
            return carry

        lax.fori_loop(0, tpw // _LANES, body, 0)
        pltpu.sync_copy(wv1, w1_hbm.at[pl.ds(base, tpw)])
        pltpu.sync_copy(wv2, w2_hbm.at[pl.ds(base, tpw)])
        pltpu.sync_copy(iv1, i1_hbm.at[pl.ds(base, tpw)])
        pltpu.sync_copy(iv2, i2_hbm.at[pl.ds(base, tpw)])

    return topk_sc


def kernel(x, W):
    t = x.shape[0] * x.shape[1]
    h = x.shape[-1]
    xt = x.reshape(t, h)
    blk = 1024
    logits, scores, scores_t = pl.pallas_call(
        _router_tc_body,
        grid=(t // blk,),
        in_specs=[
            pl.BlockSpec((blk, h), lambda i: (i, 0)),
            pl.BlockSpec((_NUM_EXPERTS, h), lambda i: (0, 0)),
        ],
        out_specs=[
            pl.BlockSpec((blk, _NUM_EXPERTS), lambda i: (i, 0)),
            pl.BlockSpec((blk, _NUM_EXPERTS), lambda i: (i, 0)),
            pl.BlockSpec((_NUM_EXPERTS, blk), lambda i: (0, i)),
        ],
        out_shape=[
            jax.ShapeDtypeStruct((t, _NUM_EXPERTS), jnp.float32),
            jax.ShapeDtypeStruct((t, _NUM_EXPERTS), jnp.float32),
            jax.ShapeDtypeStruct((_NUM_EXPERTS, t), jnp.float32),
        ],
    )(xt, W)
    w1, w2, i1, i2 = _make_topk_sc(t)(scores_t)
    expert_weights = jnp.stack([w1, w2], axis=-1)
    expert_indices = jnp.stack([i1, i2], axis=-1)
    return scores, logits, expert_weights, expert_indices


# speedup vs baseline: 1.3381x; 1.3381x over previous
import functools

import jax
import jax.numpy as jnp
from jax import lax
from jax.experimental import pallas as pl
from jax.experimental.pallas import tpu as pltpu
from jax.experimental.pallas import tpu_sc as plsc

_NUM_EXPERTS = 8
_TOP_K = 2
_LANES = 16


def _router_tc_body(x_ref, w_ref, logits_t_ref, scores_t_ref):
    l = lax.dot_general(
        x_ref[...], w_ref[...],
        dimension_numbers=(((1,), (1,)), ((), ())),
        preferred_element_type=jnp.float32,
    )
    logits_t_ref[...] = l.T
    m = jnp.max(l, axis=-1, keepdims=True)
    e = jnp.exp(l - m)
    s = e / jnp.sum(e, axis=-1, keepdims=True)
    scores_t_ref[...] = s.T


@functools.lru_cache(maxsize=None)
def _make_topk_sc(total_tokens):
    info = plsc.get_sparse_core_info()
    nc, ns = info.num_cores, info.num_subcores
    nw = nc * ns
    tpw = total_tokens // nw
    assert tpw * nw == total_tokens and tpw % _LANES == 0
    mesh = plsc.VectorSubcoreMesh(core_axis_name="c", subcore_axis_name="s")

    @functools.partial(
        pl.kernel,
        mesh=mesh,
        out_type=(
            jax.ShapeDtypeStruct((total_tokens,), jnp.float32),
            jax.ShapeDtypeStruct((total_tokens,), jnp.float32),
            jax.ShapeDtypeStruct((total_tokens,), jnp.int32),
            jax.ShapeDtypeStruct((total_tokens,), jnp.int32),
        ),
        scratch_types=[
            pltpu.VMEM((_NUM_EXPERTS * tpw,), jnp.float32),
            pltpu.VMEM((tpw,), jnp.float32),
            pltpu.VMEM((tpw,), jnp.float32),
            pltpu.VMEM((tpw,), jnp.int32),
            pltpu.VMEM((tpw,), jnp.int32),
        ],
    )
    def topk_sc(st_hbm, w1_hbm, w2_hbm, i1_hbm, i2_hbm, sv, wv1, wv2, iv1, iv2):
        wid = lax.axis_index("s") * nc + lax.axis_index("c")
        base = wid * tpw
        for e in range(_NUM_EXPERTS):
            pltpu.sync_copy(st_hbm.at[e, pl.ds(base, tpw)],
                            sv.at[pl.ds(e * tpw, tpw)])

        def body(g, carry):
            off = g * _LANES
            cols = [sv[pl.ds(e * tpw + off, _LANES)]
                    for e in range(_NUM_EXPERTS)]
            m1 = cols[0]
            i1 = jnp.zeros((_LANES,), jnp.int32)
            m2 = jnp.full((_LANES,), -jnp.inf, jnp.float32)
            i2 = jnp.zeros((_LANES,), jnp.int32)
            for e in range(1, _NUM_EXPERTS):
                c = cols[e]
                ev = jnp.full((_LANES,), e, jnp.int32)
                new_top = c > m1
                beats2 = c > m2
                m2 = jnp.where(new_top, m1, jnp.where(beats2, c, m2))
                i2 = jnp.where(new_top, i1, jnp.where(beats2, ev, i2))
                m1 = jnp.where(new_top, c, m1)
                i1 = jnp.where(new_top, ev, i1)
            wv1[pl.ds(off, _LANES)] = m1
            wv2[pl.ds(off, _LANES)] = m2
            iv1[pl.ds(off, _LANES)] = i1
            iv2[pl.ds(off, _LANES)] = i2
            return carry

        lax.fori_loop(0, tpw // _LANES, body, 0)
        pltpu.sync_copy(wv1, w1_hbm.at[pl.ds(base, tpw)])
        pltpu.sync_copy(wv2, w2_hbm.at[pl.ds(base, tpw)])
        pltpu.sync_copy(iv1, i1_hbm.at[pl.ds(base, tpw)])
        pltpu.sync_copy(iv2, i2_hbm.at[pl.ds(base, tpw)])

    return topk_sc


def kernel(x, W):
    t = x.shape[0] * x.shape[1]
    h = x.shape[-1]
    xt = x.reshape(t, h)
    blk = 4096
    logits_t, scores_t = pl.pallas_call(
        _router_tc_body,
        grid=(t // blk,),
        in_specs=[
            pl.BlockSpec((blk, h), lambda i: (i, 0)),
            pl.BlockSpec((_NUM_EXPERTS, h), lambda i: (0, 0)),
        ],
        out_specs=[
            pl.BlockSpec((_NUM_EXPERTS, blk), lambda i: (0, i)),
            pl.BlockSpec((_NUM_EXPERTS, blk), lambda i: (0, i)),
        ],
        out_shape=[
            jax.ShapeDtypeStruct((_NUM_EXPERTS, t), jnp.float32),
            jax.ShapeDtypeStruct((_NUM_EXPERTS, t), jnp.float32),
        ],
    )(xt, W)
    w1, w2, i1, i2 = _make_topk_sc(t)(scores_t)
    expert_weights = jnp.stack([w1, w2], axis=-1)
    expert_indices = jnp.stack([i1, i2], axis=-1)
    return scores_t.T, logits_t.T, expert_weights, expert_indices
